# baseline (speedup 1.0000x reference)
import functools

import jax
import jax.numpy as jnp
from jax import lax
from jax.experimental import pallas as pl
from jax.experimental.pallas import tpu as pltpu
from jax.experimental.pallas import tpu_sc as plsc

N_NODES = 10000
N_EDGES = 320000
D = 128
N_GRAPHS = 256

NC = 2
NS = 16
NW = NC * NS
E_PER_TILE = N_EDGES // NW
CHUNK = 56
N_CHUNKS = 180
E_PAD_TILE = N_CHUNKS * CHUNK
N_ACC = N_NODES + 16

ROWS_A = 640
ROWS_LAST = N_NODES - (NS - 1) * ROWS_A

RB = 1000
N_RB = N_NODES // RB

_mesh = plsc.VectorSubcoreMesh(core_axis_name="c", subcore_axis_name="s")


_NB_DEG = 5
DEG_CHUNK = 80
DEG_NCH = E_PER_TILE // DEG_CHUNK


@functools.partial(
    pl.kernel,
    out_type=jax.ShapeDtypeStruct((NC, N_NODES), jnp.float32),
    mesh=_mesh,
    scratch_types=[
        pltpu.VMEM((DEG_NCH, DEG_CHUNK), jnp.int32),
        pltpu.VMEM((DEG_CHUNK,), jnp.float32),
        pltpu.VMEM((ROWS_A,), jnp.float32),
        pltpu.VMEM_SHARED((N_NODES,), jnp.float32),
        pltpu.SemaphoreType.DMA,
    ],
)
def _deg_kernel(dst_hbm, out_hbm, di, ones_v, zeros_v, acc, sem):
    c = lax.axis_index("c")
    s = lax.axis_index("s")
    wid = c * NS + s
    for i in range(DEG_CHUNK // 16):
        ones_v[pl.ds(i * 16, 16)] = jnp.ones((16,), jnp.float32)
    for i in range(ROWS_A // 16):
        zeros_v[pl.ds(i * 16, 16)] = jnp.zeros((16,), jnp.float32)

    start = s * ROWS_A

    @pl.when(s < NS - 1)
    def _():
        pltpu.sync_copy(zeros_v, acc.at[pl.ds(start, ROWS_A)])

    @pl.when(s == NS - 1)
    def _():
        pltpu.sync_copy(zeros_v.at[pl.ds(0, ROWS_LAST)],
                        acc.at[pl.ds((NS - 1) * ROWS_A, ROWS_LAST)])

    pltpu.sync_copy(dst_hbm.at[wid], di)
    plsc.subcore_barrier()

    def body(g, carry):
        j0 = g * _NB_DEG
        for b in range(_NB_DEG):
            pltpu.async_copy(ones_v, acc.at[di.at[j0 + b]], sem, add=True)

        @pl.when(g > 0)
        def _():
            for b in range(_NB_DEG):
                pltpu.make_async_copy(ones_v, acc.at[di.at[0]], sem).wait()
        return carry

    lax.fori_loop(0, DEG_NCH // _NB_DEG, body, 0)
    for b in range(_NB_DEG):
        pltpu.make_async_copy(ones_v, acc.at[di.at[0]], sem).wait()
    plsc.subcore_barrier()

    @pl.when(s == 0)
    def _():
        pltpu.sync_copy(acc, out_hbm.at[c])


NB = 6
N_GROUPS = N_CHUNKS // NB
TAIL = N_CHUNKS - N_GROUPS * NB


@functools.partial(
    pl.kernel,
    out_type=jax.ShapeDtypeStruct((NC, N_NODES, D), jnp.float32),
    mesh=_mesh,
    scratch_types=[
        pltpu.VMEM((4 * NB, CHUNK), jnp.int32),
        pltpu.VMEM((NB * CHUNK, D), jnp.float32),
        pltpu.VMEM_SHARED((N_ACC, D), jnp.float32),
        [pltpu.SemaphoreType.DMA for _ in range(NB)],
        [pltpu.SemaphoreType.DMA for _ in range(NB)],
        [pltpu.SemaphoreType.DMA for _ in range(NB)],
    ],
)
def _edge_pass(src_hbm, dst_hbm, y_hbm, zeros_hbm, out_hbm,
               idx, rows_all, acc, isem, gsem, ssem):
    rows = [rows_all.at[pl.ds(b * CHUNK, CHUNK)] for b in range(NB)]
    c = lax.axis_index("c")
    s = lax.axis_index("s")
    wid = c * NS + s
    start = s * ROWS_A

    @pl.when(s < NS - 1)
    def _():
        pltpu.sync_copy(zeros_hbm.at[pl.ds(start, ROWS_A)],
                        acc.at[pl.ds(start, ROWS_A)])

    @pl.when(s == NS - 1)
    def _():
        pltpu.sync_copy(zeros_hbm.at[pl.ds((NS - 1) * ROWS_A, ROWS_LAST)],
                        acc.at[pl.ds((NS - 1) * ROWS_A, ROWS_LAST)])

    plsc.subcore_barrier()

    def fetch_idx(j, slot, b):
        pltpu.async_copy(src_hbm.at[wid, j], idx.at[slot], isem[b])
        pltpu.async_copy(dst_hbm.at[wid, j], idx.at[2 * NB + slot], isem[b])

    def wait_idx(slot, b):
        pltpu.make_async_copy(src_hbm.at[wid, 0], idx.at[slot],
                              isem[b]).wait()
        pltpu.make_async_copy(dst_hbm.at[wid, 0], idx.at[2 * NB + slot],
                              isem[b]).wait()

    def wait_gather(b):
        pltpu.make_async_copy(y_hbm.at[idx.at[b]], rows[b],
                              gsem[b]).wait()

    def wait_scatter(b):
        pltpu.make_async_copy(rows[b], acc.at[idx.at[2 * NB + b]], ssem[b]).wait()

    for b in range(NB):
        fetch_idx(b, b, b)

    def body(g, carry):
        j0 = g * NB
        p = (g % 2) * NB
        pn = ((g + 1) % 2) * NB
        for b in range(NB):
            wait_idx(p + b, b)

            @pl.when(g > 0)
            def _():
                wait_scatter(b)
            pltpu.async_copy(y_hbm.at[idx.at[p + b]], rows[b], gsem[b])
        for b in range(NB):
            wait_gather(b)
            pltpu.async_copy(rows[b], acc.at[idx.at[2 * NB + p + b]], ssem[b],
                             add=True)
        for b in range(NB):
            nxt = j0 + b + NB

            @pl.when(nxt < N_CHUNKS)
            def _():
                fetch_idx(nxt, pn + b, b)
        return carry

    lax.fori_loop(0, N_GROUPS, body, 0)

    for b in range(TAIL, NB):
        wait_scatter(b)
    plsc.subcore_barrier()

    @pl.when(s < NS - 1)
    def _():
        pltpu.sync_copy(acc.at[pl.ds(start, ROWS_A)],
                        out_hbm.at[c, pl.ds(start, ROWS_A)])

    @pl.when(s == NS - 1)
    def _():
        pltpu.sync_copy(acc.at[pl.ds((NS - 1) * ROWS_A, ROWS_LAST)],
                        out_hbm.at[c, pl.ds((NS - 1) * ROWS_A, ROWS_LAST)])


def _dinv_of(degp_blk):
    deg = degp_blk[0, 0, 0, :] + degp_blk[1, 0, 0, :] + 1.0
    return lax.rsqrt(deg)


def _tca_body(x_ref, w_ref, degp_ref, y_ref):
    dinv = _dinv_of(degp_ref[...])
    xw = jnp.dot(x_ref[...], w_ref[...], preferred_element_type=jnp.float32)
    y_ref[...] = xw * dinv[:, None]


def _tca(x, W1, degp):
    return pl.pallas_call(
        _tca_body,
        grid=(N_RB,),
        in_specs=[
            pl.BlockSpec((RB, D), lambda i: (i, 0)),
            pl.BlockSpec((D, D), lambda i: (0, 0)),
            pl.BlockSpec((NC, 1, 1, RB), lambda i: (0, i, 0, 0)),
        ],
        out_specs=pl.BlockSpec((RB, D), lambda i: (i, 0)),
        out_shape=jax.ShapeDtypeStruct((N_NODES, D), jnp.float32),
    )(x, W1, degp)


def _tcb_body(a_ref, y1_ref, degp_ref, b1_ref, w2_ref, y2_ref):
    dinv = _dinv_of(degp_ref[...])
    agg = a_ref[0] + a_ref[1] + y1_ref[...]
    h1 = jnp.maximum(agg * dinv[:, None] + b1_ref[...], 0.0)
    y2_ref[...] = jnp.dot(h1, w2_ref[...],
                          preferred_element_type=jnp.float32) * dinv[:, None]


def _tcb(A1, y1, degp, b1, W2):
    return pl.pallas_call(
        _tcb_body,
        grid=(N_RB,),
        in_specs=[
            pl.BlockSpec((NC, RB, D), lambda i: (0, i, 0)),
            pl.BlockSpec((RB, D), lambda i: (i, 0)),
            pl.BlockSpec((NC, 1, 1, RB), lambda i: (0, i, 0, 0)),
            pl.BlockSpec((D,), lambda i: (0,)),
            pl.BlockSpec((D, D), lambda i: (0, 0)),
        ],
        out_specs=pl.BlockSpec((RB, D), lambda i: (i, 0)),
        out_shape=jax.ShapeDtypeStruct((N_NODES, D), jnp.float32),
    )(A1, y1, degp, b1, W2)


def _tcc_body(a_ref, y2_ref, degp_ref, b2_ref, batch_ref, wfc_ref, bfc_ref,
              acc_ref, cnt_ref, out_ref):
    i = pl.program_id(0)
    dinv = _dinv_of(degp_ref[...])
    agg = a_ref[0] + a_ref[1] + y2_ref[...]
    h2 = jnp.maximum(agg * dinv[:, None] + b2_ref[...], 0.0)

    b = batch_ref[0, 0, :]
    gids = lax.broadcasted_iota(jnp.int32, (RB, N_GRAPHS), 1)
    oh = (b[:, None] == gids).astype(jnp.float32)
    part = lax.dot_general(oh, h2, (((0,), (0,)), ((), ())),
                           preferred_element_type=jnp.float32)
    ones = jnp.ones((RB, D), jnp.float32)
    pcnt = lax.dot_general(oh, ones, (((0,), (0,)), ((), ())),
                           preferred_element_type=jnp.float32)

    @pl.when(i == 0)
    def _():
        acc_ref[...] = jnp.zeros_like(acc_ref)
        cnt_ref[...] = jnp.zeros_like(cnt_ref)

    acc_ref[...] += part
    cnt_ref[...] += pcnt

    @pl.when(i == pl.num_programs(0) - 1)
    def _():
        pooled = acc_ref[...] / jnp.maximum(cnt_ref[...], 1.0)
        out_ref[...] = jnp.dot(pooled, wfc_ref[...],
                               preferred_element_type=jnp.float32) + bfc_ref[...]


def _tcc(A2, y2, degp, b2, batch3, Wfc, bfc):
    _, _, out = pl.pallas_call(
        _tcc_body,
        grid=(N_RB,),
        in_specs=[
            pl.BlockSpec((NC, RB, D), lambda i: (0, i, 0)),
            pl.BlockSpec((RB, D), lambda i: (i, 0)),
            pl.BlockSpec((NC, 1, 1, RB), lambda i: (0, i, 0, 0)),
            pl.BlockSpec((D,), lambda i: (0,)),
            pl.BlockSpec((1, 1, RB), lambda i: (i, 0, 0)),
            pl.BlockSpec((D, D), lambda i: (0, 0)),
            pl.BlockSpec((D,), lambda i: (0,)),
        ],
        out_specs=[
            pl.BlockSpec((N_GRAPHS, D), lambda i: (0, 0)),
            pl.BlockSpec((N_GRAPHS, D), lambda i: (0, 0)),
            pl.BlockSpec((N_GRAPHS, D), lambda i: (0, 0)),
        ],
        out_shape=[
            jax.ShapeDtypeStruct((N_GRAPHS, D), jnp.float32),
            jax.ShapeDtypeStruct((N_GRAPHS, D), jnp.float32),
            jax.ShapeDtypeStruct((N_GRAPHS, D), jnp.float32),
        ],
    )(A2, y2, degp, b2, batch3, Wfc, bfc)
    return out


def kernel(x, edge_index, batch, W1, b1, W2, b2, Wfc, bfc):
    pad_e = E_PAD_TILE - E_PER_TILE
    src = jnp.concatenate(
        [edge_index[0].reshape(NW, E_PER_TILE),
         jnp.zeros((NW, pad_e), jnp.int32)], axis=1
    ).reshape(NW, N_CHUNKS, CHUNK)
    pad_dst = N_NODES + (jnp.arange(pad_e, dtype=jnp.int32) % 16)
    dst = jnp.concatenate(
        [edge_index[1].reshape(NW, E_PER_TILE),
         jnp.broadcast_to(pad_dst, (NW, pad_e))], axis=1
    ).reshape(NW, N_CHUNKS, CHUNK)
    zeros_nd = jnp.zeros((N_NODES, D), jnp.float32)

    dst_deg = edge_index[1].reshape(NW, DEG_NCH, DEG_CHUNK)
    degp = _deg_kernel(dst_deg)
    degp4 = degp.reshape(NC, N_RB, 1, RB)
    y1 = _tca(x, W1, degp4)
    A1 = _edge_pass(src, dst, y1, zeros_nd)
    y2 = _tcb(A1, y1, degp4, b1, W2)
    A2 = _edge_pass(src, dst, y2, zeros_nd)
    batch3 = batch.reshape(N_RB, 1, RB)
    return _tcc(A2, y2, degp4, b2, batch3, Wfc, bfc)


# speedup vs baseline: 1.0019x; 1.0019x over previous
import functools

import jax
import jax.numpy as jnp
from jax import lax
from jax.experimental import pallas as pl
from jax.experimental.pallas import tpu as pltpu
from jax.experimental.pallas import tpu_sc as plsc

N_NODES = 10000
N_EDGES = 320000
D = 128
N_GRAPHS = 256

NC = 2
NS = 16
NW = NC * NS
E_PER_TILE = N_EDGES // NW
CHUNK = 56
N_CHUNKS = 180
E_PAD_TILE = N_CHUNKS * CHUNK
N_ACC = N_NODES + 16

ROWS_A = 640
ROWS_LAST = N_NODES - (NS - 1) * ROWS_A

RB = 1000
N_RB = N_NODES // RB

_mesh = plsc.VectorSubcoreMesh(core_axis_name="c", subcore_axis_name="s")


_NB_DEG = 5
DEG_CHUNK = 80
DEG_NCH = E_PER_TILE // DEG_CHUNK


@functools.partial(
    pl.kernel,
    out_type=jax.ShapeDtypeStruct((NC, N_NODES), jnp.float32),
    mesh=_mesh,
    scratch_types=[
        pltpu.VMEM((DEG_NCH, DEG_CHUNK), jnp.int32),
        pltpu.VMEM((DEG_CHUNK,), jnp.float32),
        pltpu.VMEM((ROWS_A,), jnp.float32),
        pltpu.VMEM_SHARED((N_NODES,), jnp.float32),
        pltpu.SemaphoreType.DMA,
    ],
)
def _deg_kernel(dst_hbm, out_hbm, di, ones_v, zeros_v, acc, sem):
    c = lax.axis_index("c")
    s = lax.axis_index("s")
    wid = c * NS + s
    for i in range(DEG_CHUNK // 16):
        ones_v[pl.ds(i * 16, 16)] = jnp.ones((16,), jnp.float32)
    for i in range(ROWS_A // 16):
        zeros_v[pl.ds(i * 16, 16)] = jnp.zeros((16,), jnp.float32)

    start = s * ROWS_A

    @pl.when(s < NS - 1)
    def _():
        pltpu.sync_copy(zeros_v, acc.at[pl.ds(start, ROWS_A)])

    @pl.when(s == NS - 1)
    def _():
        pltpu.sync_copy(zeros_v.at[pl.ds(0, ROWS_LAST)],
                        acc.at[pl.ds((NS - 1) * ROWS_A, ROWS_LAST)])

    pltpu.sync_copy(dst_hbm.at[wid], di)
    plsc.subcore_barrier()

    def body(g, carry):
        j0 = g * _NB_DEG
        for b in range(_NB_DEG):
            pltpu.async_copy(ones_v, acc.at[di.at[j0 + b]], sem, add=True)

        @pl.when(g > 0)
        def _():
            for b in range(_NB_DEG):
                pltpu.make_async_copy(ones_v, acc.at[di.at[0]], sem).wait()
        return carry

    lax.fori_loop(0, DEG_NCH // _NB_DEG, body, 0)
    for b in range(_NB_DEG):
        pltpu.make_async_copy(ones_v, acc.at[di.at[0]], sem).wait()
    plsc.subcore_barrier()

    @pl.when(s == 0)
    def _():
        pltpu.sync_copy(acc, out_hbm.at[c])


NB = 6
N_GROUPS = N_CHUNKS // NB
TAIL = N_CHUNKS - N_GROUPS * NB


@functools.partial(
    pl.kernel,
    out_type=jax.ShapeDtypeStruct((NC, N_NODES, D), jnp.float32),
    mesh=_mesh,
    scratch_types=[
        pltpu.VMEM((2 * NB, CHUNK), jnp.int32),
        pltpu.VMEM((2 * NB, CHUNK), jnp.int32),
        pltpu.VMEM((NB * CHUNK, D), jnp.float32),
        pltpu.VMEM_SHARED((N_ACC, D), jnp.float32),
        [pltpu.SemaphoreType.DMA for _ in range(NB)],
        [pltpu.SemaphoreType.DMA for _ in range(NB)],
        [pltpu.SemaphoreType.DMA for _ in range(NB)],
    ],
)
def _edge_pass(src_hbm, dst_hbm, y_hbm, zeros_hbm, out_hbm,
               si_r, di_r, rows_all, acc, isem, gsem, ssem):
    rows = [rows_all.at[pl.ds(b * CHUNK, CHUNK)] for b in range(NB)]
    c = lax.axis_index("c")
    s = lax.axis_index("s")
    wid = c * NS + s
    start = s * ROWS_A

    @pl.when(s < NS - 1)
    def _():
        pltpu.sync_copy(zeros_hbm.at[pl.ds(start, ROWS_A)],
                        acc.at[pl.ds(start, ROWS_A)])

    @pl.when(s == NS - 1)
    def _():
        pltpu.sync_copy(zeros_hbm.at[pl.ds((NS - 1) * ROWS_A, ROWS_LAST)],
                        acc.at[pl.ds((NS - 1) * ROWS_A, ROWS_LAST)])

    plsc.subcore_barrier()

    def fetch_idx(j, slot, b):
        pltpu.async_copy(src_hbm.at[wid, j], si_r.at[slot], isem[b])
        pltpu.async_copy(dst_hbm.at[wid, j], di_r.at[slot], isem[b])

    def wait_idx(slot, b):
        pltpu.make_async_copy(src_hbm.at[wid, 0], si_r.at[slot],
                              isem[b]).wait()
        pltpu.make_async_copy(dst_hbm.at[wid, 0], di_r.at[slot],
                              isem[b]).wait()

    def wait_gather(b):
        pltpu.make_async_copy(y_hbm.at[si_r.at[b]], rows[b],
                              gsem[b]).wait()

    def wait_scatter(b):
        pltpu.make_async_copy(rows[b], acc.at[di_r.at[b]], ssem[b]).wait()

    for b in range(NB):
        fetch_idx(b, b, b)

    def body(g, carry):
        j0 = g * NB
        p = (g % 2) * NB
        pn = ((g + 1) % 2) * NB
        for b in range(NB):
            wait_idx(p + b, b)

            @pl.when(g > 0)
            def _():
                wait_scatter(b)
            pltpu.async_copy(y_hbm.at[si_r.at[p + b]], rows[b], gsem[b])
        for b in range(NB):
            wait_gather(b)
            pltpu.async_copy(rows[b], acc.at[di_r.at[p + b]], ssem[b],
                             add=True)
        for b in range(NB):
            nxt = j0 + b + NB

            @pl.when(nxt < N_CHUNKS)
            def _():
                fetch_idx(nxt, pn + b, b)
        return carry

    lax.fori_loop(0, N_GROUPS, body, 0)

    for b in range(TAIL, NB):
        wait_scatter(b)
    plsc.subcore_barrier()

    @pl.when(s < NS - 1)
    def _():
        pltpu.sync_copy(acc.at[pl.ds(start, ROWS_A)],
                        out_hbm.at[c, pl.ds(start, ROWS_A)])

    @pl.when(s == NS - 1)
    def _():
        pltpu.sync_copy(acc.at[pl.ds((NS - 1) * ROWS_A, ROWS_LAST)],
                        out_hbm.at[c, pl.ds((NS - 1) * ROWS_A, ROWS_LAST)])


def _dinv_of(degp_blk):
    deg = degp_blk[0, 0, 0, :] + degp_blk[1, 0, 0, :] + 1.0
    return lax.rsqrt(deg)


def _tca_body(x_ref, w_ref, degp_ref, y_ref):
    dinv = _dinv_of(degp_ref[...])
    xw = jnp.dot(x_ref[...], w_ref[...], preferred_element_type=jnp.float32)
    y_ref[...] = xw * dinv[:, None]


def _tca(x, W1, degp):
    return pl.pallas_call(
        _tca_body,
        grid=(N_RB,),
        in_specs=[
            pl.BlockSpec((RB, D), lambda i: (i, 0)),
            pl.BlockSpec((D, D), lambda i: (0, 0)),
            pl.BlockSpec((NC, 1, 1, RB), lambda i: (0, i, 0, 0)),
        ],
        out_specs=pl.BlockSpec((RB, D), lambda i: (i, 0)),
        out_shape=jax.ShapeDtypeStruct((N_NODES, D), jnp.float32),
    )(x, W1, degp)


def _tcb_body(a_ref, y1_ref, degp_ref, b1_ref, w2_ref, y2_ref):
    dinv = _dinv_of(degp_ref[...])
    agg = a_ref[0] + a_ref[1] + y1_ref[...]
    h1 = jnp.maximum(agg * dinv[:, None] + b1_ref[...], 0.0)
    y2_ref[...] = jnp.dot(h1, w2_ref[...],
                          preferred_element_type=jnp.float32) * dinv[:, None]


def _tcb(A1, y1, degp, b1, W2):
    return pl.pallas_call(
        _tcb_body,
        grid=(N_RB,),
        in_specs=[
            pl.BlockSpec((NC, RB, D), lambda i: (0, i, 0)),
            pl.BlockSpec((RB, D), lambda i: (i, 0)),
            pl.BlockSpec((NC, 1, 1, RB), lambda i: (0, i, 0, 0)),
            pl.BlockSpec((D,), lambda i: (0,)),
            pl.BlockSpec((D, D), lambda i: (0, 0)),
        ],
        out_specs=pl.BlockSpec((RB, D), lambda i: (i, 0)),
        out_shape=jax.ShapeDtypeStruct((N_NODES, D), jnp.float32),
    )(A1, y1, degp, b1, W2)


def _tcc_body(a_ref, y2_ref, degp_ref, b2_ref, batch_ref, wfc_ref, bfc_ref,
              acc_ref, cnt_ref, out_ref):
    i = pl.program_id(0)
    dinv = _dinv_of(degp_ref[...])
    agg = a_ref[0] + a_ref[1] + y2_ref[...]
    h2 = jnp.maximum(agg * dinv[:, None] + b2_ref[...], 0.0)

    b = batch_ref[0, 0, :]
    gids = lax.broadcasted_iota(jnp.int32, (RB, N_GRAPHS), 1)
    oh = (b[:, None] == gids).astype(jnp.float32)
    part = lax.dot_general(oh, h2, (((0,), (0,)), ((), ())),
                           preferred_element_type=jnp.float32)
    ones = jnp.ones((RB, D), jnp.float32)
    pcnt = lax.dot_general(oh, ones, (((0,), (0,)), ((), ())),
                           preferred_element_type=jnp.float32)

    @pl.when(i == 0)
    def _():
        acc_ref[...] = jnp.zeros_like(acc_ref)
        cnt_ref[...] = jnp.zeros_like(cnt_ref)

    acc_ref[...] += part
    cnt_ref[...] += pcnt

    @pl.when(i == pl.num_programs(0) - 1)
    def _():
        pooled = acc_ref[...] / jnp.maximum(cnt_ref[...], 1.0)
        out_ref[...] = jnp.dot(pooled, wfc_ref[...],
                               preferred_element_type=jnp.float32) + bfc_ref[...]


def _tcc(A2, y2, degp, b2, batch3, Wfc, bfc):
    _, _, out = pl.pallas_call(
        _tcc_body,
        grid=(N_RB,),
        in_specs=[
            pl.BlockSpec((NC, RB, D), lambda i: (0, i, 0)),
            pl.BlockSpec((RB, D), lambda i: (i, 0)),
            pl.BlockSpec((NC, 1, 1, RB), lambda i: (0, i, 0, 0)),
            pl.BlockSpec((D,), lambda i: (0,)),
            pl.BlockSpec((1, 1, RB), lambda i: (i, 0, 0)),
            pl.BlockSpec((D, D), lambda i: (0, 0)),
            pl.BlockSpec((D,), lambda i: (0,)),
        ],
        out_specs=[
            pl.BlockSpec((N_GRAPHS, D), lambda i: (0, 0)),
            pl.BlockSpec((N_GRAPHS, D), lambda i: (0, 0)),
            pl.BlockSpec((N_GRAPHS, D), lambda i: (0, 0)),
        ],
        out_shape=[
            jax.ShapeDtypeStruct((N_GRAPHS, D), jnp.float32),
            jax.ShapeDtypeStruct((N_GRAPHS, D), jnp.float32),
            jax.ShapeDtypeStruct((N_GRAPHS, D), jnp.float32),
        ],
    )(A2, y2, degp, b2, batch3, Wfc, bfc)
    return out


def kernel(x, edge_index, batch, W1, b1, W2, b2, Wfc, bfc):
    pad_e = E_PAD_TILE - E_PER_TILE
    src = jnp.concatenate(
        [edge_index[0].reshape(NW, E_PER_TILE),
         jnp.zeros((NW, pad_e), jnp.int32)], axis=1
    ).reshape(NW, N_CHUNKS, CHUNK)
    pad_dst = N_NODES + (jnp.arange(pad_e, dtype=jnp.int32) % 16)
    dst = jnp.concatenate(
        [edge_index[1].reshape(NW, E_PER_TILE),
         jnp.broadcast_to(pad_dst, (NW, pad_e))], axis=1
    ).reshape(NW, N_CHUNKS, CHUNK)
    zeros_nd = jnp.zeros((N_NODES, D), jnp.float32)

    dst_deg = edge_index[1].reshape(NW, DEG_NCH, DEG_CHUNK)
    degp = _deg_kernel(dst_deg)
    degp4 = degp.reshape(NC, N_RB, 1, RB)
    y1 = _tca(x, W1, degp4)
    A1 = _edge_pass(src, dst, y1, zeros_nd)
    y2 = _tcb(A1, y1, degp4, b1, W2)
    A2 = _edge_pass(src, dst, y2, zeros_nd)
    batch3 = batch.reshape(N_RB, 1, RB)
    return _tcc(A2, y2, degp4, b2, batch3, Wfc, bfc)


# speedup vs baseline: 1.5277x; 1.5249x over previous
import functools

import jax
import jax.numpy as jnp
from jax import lax
from jax.experimental import pallas as pl
from jax.experimental.pallas import tpu as pltpu
from jax.experimental.pallas import tpu_sc as plsc

N_NODES = 10000
N_EDGES = 320000
D = 128
N_GRAPHS = 256

NC = 2
NS = 16
NW = NC * NS
E_PER_TILE = N_EDGES // NW
CHUNK = 80
N_CHUNKS = E_PER_TILE // CHUNK

ROWS_A = 640
ROWS_LAST = N_NODES - (NS - 1) * ROWS_A

RB = 1000
N_RB = N_NODES // RB

_mesh = plsc.VectorSubcoreMesh(core_axis_name="c", subcore_axis_name="s")


_NB_DEG = 5


@functools.partial(
    pl.kernel,
    out_type=jax.ShapeDtypeStruct((NC, N_NODES), jnp.float32),
    mesh=_mesh,
    scratch_types=[
        pltpu.VMEM((125, CHUNK), jnp.int32),
        pltpu.VMEM((CHUNK,), jnp.float32),
        pltpu.VMEM((ROWS_A,), jnp.float32),
        pltpu.VMEM_SHARED((N_NODES,), jnp.float32),
        pltpu.SemaphoreType.DMA,
    ],
)
def _deg_kernel(dst_hbm, out_hbm, di, ones_v, zeros_v, acc, sem):
    c = lax.axis_index("c")
    s = lax.axis_index("s")
    wid = c * NS + s
    for i in range(CHUNK // 16):
        ones_v[pl.ds(i * 16, 16)] = jnp.ones((16,), jnp.float32)
    for i in range(ROWS_A // 16):
        zeros_v[pl.ds(i * 16, 16)] = jnp.zeros((16,), jnp.float32)

    start = s * ROWS_A

    @pl.when(s < NS - 1)
    def _():
        pltpu.sync_copy(zeros_v, acc.at[pl.ds(start, ROWS_A)])

    @pl.when(s == NS - 1)
    def _():
        pltpu.sync_copy(zeros_v.at[pl.ds(0, ROWS_LAST)],
                        acc.at[pl.ds((NS - 1) * ROWS_A, ROWS_LAST)])

    pltpu.sync_copy(dst_hbm.at[wid], di)
    plsc.subcore_barrier()

    def body(g, carry):
        j0 = g * _NB_DEG
        for b in range(_NB_DEG):
            pltpu.async_copy(ones_v, acc.at[di.at[j0 + b]], sem, add=True)

        @pl.when(g > 0)
        def _():
            for b in range(_NB_DEG):
                pltpu.make_async_copy(ones_v, acc.at[di.at[0]], sem).wait()
        return carry

    lax.fori_loop(0, N_CHUNKS // _NB_DEG, body, 0)
    for b in range(_NB_DEG):
        pltpu.make_async_copy(ones_v, acc.at[di.at[0]], sem).wait()
    plsc.subcore_barrier()

    @pl.when(s == 0)
    def _():
        pltpu.sync_copy(acc, out_hbm.at[c])


NB = 4
N_GROUPS = N_CHUNKS // NB
TAIL = N_CHUNKS - N_GROUPS * NB


@functools.partial(
    pl.kernel,
    out_type=jax.ShapeDtypeStruct((NC, N_NODES, D), jnp.float32),
    mesh=_mesh,
    scratch_types=[
        pltpu.VMEM((2 * NB, CHUNK), jnp.int32),
        pltpu.VMEM((2 * NB, CHUNK), jnp.int32),
        [pltpu.VMEM((CHUNK, D), jnp.float32) for _ in range(NB)],
        pltpu.VMEM_SHARED((N_NODES, D), jnp.float32),
        [pltpu.SemaphoreType.DMA for _ in range(NB)],
        [pltpu.SemaphoreType.DMA for _ in range(NB)],
        [pltpu.SemaphoreType.DMA for _ in range(NB)],
    ],
)
def _edge_pass(src_hbm, dst_hbm, y_hbm, zeros_hbm, out_hbm,
               si, di, rows, acc, isem, gsem, ssem):
    c = lax.axis_index("c")
    s = lax.axis_index("s")
    wid = c * NS + s
    start = s * ROWS_A

    @pl.when(s < NS - 1)
    def _():
        pltpu.sync_copy(zeros_hbm.at[pl.ds(start, ROWS_A)],
                        acc.at[pl.ds(start, ROWS_A)])

    @pl.when(s == NS - 1)
    def _():
        pltpu.sync_copy(zeros_hbm.at[pl.ds((NS - 1) * ROWS_A, ROWS_LAST)],
                        acc.at[pl.ds((NS - 1) * ROWS_A, ROWS_LAST)])

    plsc.subcore_barrier()

    def fetch_idx(j, slot, b):
        pltpu.async_copy(src_hbm.at[wid, j], si.at[slot], isem[b])
        pltpu.async_copy(dst_hbm.at[wid, j], di.at[slot], isem[b])

    def wait_idx(slot, b):
        pltpu.make_async_copy(src_hbm.at[wid, 0], si.at[slot], isem[b]).wait()
        pltpu.make_async_copy(dst_hbm.at[wid, 0], di.at[slot], isem[b]).wait()

    def wait_gather(b):
        pltpu.make_async_copy(y_hbm.at[si.at[b]], rows[b], gsem[b]).wait()

    def wait_scatter(b):
        pltpu.make_async_copy(rows[b], acc.at[di.at[b]], ssem[b]).wait()

    for b in range(NB):
        fetch_idx(b, b, b)

    def body(g, carry):
        j0 = g * NB
        p = (g % 2) * NB
        pn = ((g + 1) % 2) * NB
        for b in range(NB):
            wait_idx(p + b, b)

            @pl.when(g > 0)
            def _():
                wait_scatter(b)
            pltpu.async_copy(y_hbm.at[si.at[p + b]], rows[b], gsem[b])
        for b in range(NB):
            wait_gather(b)
            pltpu.async_copy(rows[b], acc.at[di.at[p + b]], ssem[b],
                             add=True)
        for b in range(NB):
            nxt = j0 + b + NB

            @pl.when(nxt < N_CHUNKS)
            def _():
                fetch_idx(nxt, pn + b, b)
        return carry

    lax.fori_loop(0, N_GROUPS, body, 0)

    pt = (N_GROUPS % 2) * NB
    for t in range(TAIL):
        wait_idx(pt + t, t)
        wait_scatter(t)
        pltpu.async_copy(y_hbm.at[si.at[pt + t]], rows[t], gsem[t])
        wait_gather(t)
        pltpu.async_copy(rows[t], acc.at[di.at[pt + t]], ssem[t], add=True)
        wait_scatter(t)
    for b in range(TAIL, NB):
        wait_scatter(b)
    plsc.subcore_barrier()

    @pl.when(s < NS - 1)
    def _():
        pltpu.sync_copy(acc.at[pl.ds(start, ROWS_A)],
                        out_hbm.at[c, pl.ds(start, ROWS_A)])

    @pl.when(s == NS - 1)
    def _():
        pltpu.sync_copy(acc.at[pl.ds((NS - 1) * ROWS_A, ROWS_LAST)],
                        out_hbm.at[c, pl.ds((NS - 1) * ROWS_A, ROWS_LAST)])


def _dinv_of(degp_blk):
    deg = degp_blk[0, 0, 0, :] + degp_blk[1, 0, 0, :] + 1.0
    return lax.rsqrt(deg)


def _tca_body(x_ref, w_ref, degp_ref, y_ref):
    dinv = _dinv_of(degp_ref[...])
    xw = jnp.dot(x_ref[...], w_ref[...], preferred_element_type=jnp.float32)
    y_ref[...] = xw * dinv[:, None]


def _tca(x, W1, degp):
    return pl.pallas_call(
        _tca_body,
        grid=(N_RB,),
        in_specs=[
            pl.BlockSpec((RB, D), lambda i: (i, 0)),
            pl.BlockSpec((D, D), lambda i: (0, 0)),
            pl.BlockSpec((NC, 1, 1, RB), lambda i: (0, i, 0, 0)),
        ],
        out_specs=pl.BlockSpec((RB, D), lambda i: (i, 0)),
        out_shape=jax.ShapeDtypeStruct((N_NODES, D), jnp.float32),
    )(x, W1, degp)


def _tcb_body(a_ref, y1_ref, degp_ref, b1_ref, w2_ref, y2_ref):
    dinv = _dinv_of(degp_ref[...])
    agg = a_ref[0] + a_ref[1] + y1_ref[...]
    h1 = jnp.maximum(agg * dinv[:, None] + b1_ref[...], 0.0)
    y2_ref[...] = jnp.dot(h1, w2_ref[...],
                          preferred_element_type=jnp.float32) * dinv[:, None]


def _tcb(A1, y1, degp, b1, W2):
    return pl.pallas_call(
        _tcb_body,
        grid=(N_RB,),
        in_specs=[
            pl.BlockSpec((NC, RB, D), lambda i: (0, i, 0)),
            pl.BlockSpec((RB, D), lambda i: (i, 0)),
            pl.BlockSpec((NC, 1, 1, RB), lambda i: (0, i, 0, 0)),
            pl.BlockSpec((D,), lambda i: (0,)),
            pl.BlockSpec((D, D), lambda i: (0, 0)),
        ],
        out_specs=pl.BlockSpec((RB, D), lambda i: (i, 0)),
        out_shape=jax.ShapeDtypeStruct((N_NODES, D), jnp.float32),
    )(A1, y1, degp, b1, W2)


def _tcc_body(a_ref, y2_ref, degp_ref, b2_ref, batch_ref, wfc_ref, bfc_ref,
              acc_ref, cnt_ref, out_ref):
    i = pl.program_id(0)
    dinv = _dinv_of(degp_ref[...])
    agg = a_ref[0] + a_ref[1] + y2_ref[...]
    h2 = jnp.maximum(agg * dinv[:, None] + b2_ref[...], 0.0)

    b = batch_ref[0, 0, :]
    gids = lax.broadcasted_iota(jnp.int32, (RB, N_GRAPHS), 1)
    oh = (b[:, None] == gids).astype(jnp.float32)
    part = lax.dot_general(oh, h2, (((0,), (0,)), ((), ())),
                           preferred_element_type=jnp.float32)
    ones = jnp.ones((RB, D), jnp.float32)
    pcnt = lax.dot_general(oh, ones, (((0,), (0,)), ((), ())),
                           preferred_element_type=jnp.float32)

    @pl.when(i == 0)
    def _():
        acc_ref[...] = jnp.zeros_like(acc_ref)
        cnt_ref[...] = jnp.zeros_like(cnt_ref)

    acc_ref[...] += part
    cnt_ref[...] += pcnt

    @pl.when(i == pl.num_programs(0) - 1)
    def _():
        pooled = acc_ref[...] / jnp.maximum(cnt_ref[...], 1.0)
        out_ref[...] = jnp.dot(pooled, wfc_ref[...],
                               preferred_element_type=jnp.float32) + bfc_ref[...]


def _tcc(A2, y2, degp, b2, batch3, Wfc, bfc):
    _, _, out = pl.pallas_call(
        _tcc_body,
        grid=(N_RB,),
        in_specs=[
            pl.BlockSpec((NC, RB, D), lambda i: (0, i, 0)),
            pl.BlockSpec((RB, D), lambda i: (i, 0)),
            pl.BlockSpec((NC, 1, 1, RB), lambda i: (0, i, 0, 0)),
            pl.BlockSpec((D,), lambda i: (0,)),
            pl.BlockSpec((1, 1, RB), lambda i: (i, 0, 0)),
            pl.BlockSpec((D, D), lambda i: (0, 0)),
            pl.BlockSpec((D,), lambda i: (0,)),
        ],
        out_specs=[
            pl.BlockSpec((N_GRAPHS, D), lambda i: (0, 0)),
            pl.BlockSpec((N_GRAPHS, D), lambda i: (0, 0)),
            pl.BlockSpec((N_GRAPHS, D), lambda i: (0, 0)),
        ],
        out_shape=[
            jax.ShapeDtypeStruct((N_GRAPHS, D), jnp.float32),
            jax.ShapeDtypeStruct((N_GRAPHS, D), jnp.float32),
            jax.ShapeDtypeStruct((N_GRAPHS, D), jnp.float32),
        ],
    )(A2, y2, degp, b2, batch3, Wfc, bfc)
    return out


def kernel(x, edge_index, batch, W1, b1, W2, b2, Wfc, bfc):
    src = edge_index[0].reshape(NW, N_CHUNKS, CHUNK)
    dst = edge_index[1].reshape(NW, N_CHUNKS, CHUNK)
    zeros_nd = jnp.zeros((N_NODES, D), jnp.float32)

    degp = _deg_kernel(dst)
    degp4 = degp.reshape(NC, N_RB, 1, RB)
    y1 = _tca(x, W1, degp4)
    A1 = _edge_pass(src, dst, y1, zeros_nd)
    y2 = _tcb(A1, y1, degp4, b1, W2)
    A2 = _edge_pass(src, dst, y2, zeros_nd)
    batch3 = batch.reshape(N_RB, 1, RB)
    return _tcc(A2, y2, degp4, b2, batch3, Wfc, bfc)
